# baseline (speedup 1.0000x reference)
import functools

import jax
import jax.numpy as jnp
from jax import lax
from jax.experimental import pallas as pl
from jax.experimental.pallas import tpu as pltpu
from jax.experimental.pallas import tpu_sc as plsc

N = 50000
E = 800000
IN_CH = 128
HID_CH = 128
OUT_CH = 64
K = 10
ALPHA = 0.1

NC = 2
NS = 16
CH = 128
CPS = 400
EPAD = NS * CPS * CH
S = 50176
FH = 32
RPS = S // NS
UCH = 112
NUC = RPS // UCH
NB = 5
NI = 10

_MESH = plsc.VectorSubcoreMesh(
    core_axis_name="c", subcore_axis_name="s", num_cores=NC, num_subcores=NS
)


@functools.partial(
    pl.kernel,
    out_type=jax.ShapeDtypeStruct((NC * S, FH), jnp.float32),
    mesh=_MESH,
    scratch_types=[
        pltpu.VMEM_SHARED((S, FH), jnp.float32),
        pltpu.VMEM((NB, CH, FH), jnp.float32),
        pltpu.VMEM((NI, 2, CH), jnp.int32),
        pltpu.VMEM((RPS,), jnp.float32),
        [pltpu.SemaphoreType.DMA] * NB,
        [pltpu.SemaphoreType.DMA] * NB,
        [pltpu.SemaphoreType.DMA] * NI,
        pltpu.SemaphoreType.DMA,
        [pltpu.SemaphoreType.DMA] * 2,
        [pltpu.SemaphoreType.DMA] * 2,
    ],
    compiler_params=pltpu.CompilerParams(
        use_tc_tiling_on_sc=False, needs_layout_passes=False
    ),
)
def _sc_appnp(h_hbm, idx_hbm, zeros_hbm, out_hbm, acc, gbuf, ibuf,
              dbuf, gsem, ssem, isem, zsem, osem, hsem):
    c = lax.axis_index("c")
    s = lax.axis_index("s")
    kbase = (c * NS + s) * CPS
    rbase = s * RPS
    obase = c * S + s * RPS

    def idx_copy(chunk, m):
        return pltpu.make_async_copy(idx_hbm.at[kbase + chunk], ibuf.at[m],
                                     isem[m])

    def zero_acc():
        return pltpu.make_async_copy(
            zeros_hbm.at[pl.ds(rbase, RPS)], acc.at[pl.ds(rbase, RPS)], zsem
        )

    def scat_wait(b, m):
        pltpu.make_async_copy(gbuf.at[b], acc.at[ibuf.at[m, 1]],
                              ssem[b]).wait()

    zero_acc().start()
    @pl.loop(0, CH)
    def _(i):
        gbuf[NB - 1, i, pl.ds(0, 16)] = jnp.full((16,), 1.0, jnp.float32)
        gbuf[NB - 1, i, pl.ds(16, 16)] = jnp.full((16,), 1.0, jnp.float32)

    for m in range(NI - 1):
        idx_copy(m, m).start()
    zero_acc().wait()
    plsc.subcore_barrier()

    @pl.loop(0, CPS, step=NI)
    def _(j):
        for b8 in range(NI):
            ch = j + b8
            db = b8 % NB
            pb = (b8 + NB - 1) % NB
            pi = (b8 + NI - 1) % NI
            idx_copy(ch, b8).wait()
            pltpu.async_copy(gbuf.at[NB - 1], acc.at[ibuf.at[b8, 1]],
                             ssem[db], add=True)

            @pl.when(ch > 0)
            def _():
                pltpu.make_async_copy(gbuf.at[NB - 1],
                                      acc.at[ibuf.at[pi, 1]],
                                      ssem[pb]).wait()

            @pl.when(ch + NI - 1 < CPS)
            def _():
                idx_copy(ch + NI - 1, pi).start()

    pltpu.make_async_copy(gbuf.at[NB - 1],
                          acc.at[ibuf.at[(CPS - 1) % NI, 1]],
                          ssem[(CPS - 1) % NB]).wait()
    plsc.subcore_barrier()

    @pl.loop(0, NUC)
    def _(t):
        pltpu.sync_copy(acc.at[pl.ds(rbase + t * UCH, UCH)],
                        gbuf.at[0, pl.ds(0, UCH)])

        @pl.loop(0, UCH // 16)
        def _(g):
            rows = g * 16 + lax.iota(jnp.int32, 16)
            dg = plsc.load_gather(
                gbuf, [jnp.zeros((16,), jnp.int32), rows,
                       jnp.zeros((16,), jnp.int32)]
            )
            dbuf[pl.ds(t * UCH + g * 16, 16)] = jnp.where(
                dg > 0.0, (1.0 - ALPHA) / dg, 0.0
            )

    zero_acc().start()
    zero_acc().wait()
    plsc.subcore_barrier()

    def sweep(src):
        def gath(chunk, b, m):
            return pltpu.make_async_copy(src.at[ibuf.at[m, 0]], gbuf.at[b],
                                         gsem[b])

        for m in range(NI - 1):
            idx_copy(m, m).start()
        for b in range(NB - 1):
            idx_copy(b, b).wait()
            gath(b, b, b).start()

        @pl.loop(0, CPS, step=NI)
        def _(j):
            for b8 in range(NI):
                ch = j + b8
                db = b8 % NB
                nb = (b8 + NB - 1) % NB
                ni = (b8 + NB - 1) % NI
                pi = (b8 + NI - 1) % NI

                gath(ch, db, b8).wait()
                pltpu.async_copy(gbuf.at[db], acc.at[ibuf.at[b8, 1]],
                                 ssem[db], add=True)

                nxt = ch + NB - 1
                @pl.when(nxt < CPS)
                def _():
                    @pl.when(ch > 0)
                    def _():
                        scat_wait(nb, pi)
                    idx_copy(nxt, ni).wait()
                    gath(nxt, nb, ni).start()

                @pl.when(ch + NI - 1 < CPS)
                def _():
                    idx_copy(ch + NI - 1, pi).start()

        for t in range(NB):
            chunk = CPS - NB + t
            scat_wait(chunk % NB, chunk % NI)
        plsc.subcore_barrier()

    def out_copy(t, tb):
        return pltpu.make_async_copy(
            gbuf.at[tb, pl.ds(0, UCH)],
            out_hbm.at[pl.ds(obase + t * UCH, UCH)], osem[tb]
        )

    def stage(t, tb):
        pltpu.sync_copy(acc.at[pl.ds(rbase + t * UCH, UCH)],
                        gbuf.at[tb, pl.ds(0, UCH)])
        pltpu.make_async_copy(
            h_hbm.at[pl.ds(obase + t * UCH, UCH)],
            gbuf.at[2 + tb, pl.ds(0, UCH)], hsem[tb]
        ).start()

    def update_sweep():
        stage(0, 0)

        @pl.loop(0, NUC, step=2)
        def _(t0):
            for b2 in range(2):
                t = t0 + b2
                tb = b2
                ob = 1 - b2

                @pl.when(t + 1 < NUC)
                def _():
                    @pl.when(t >= 1)
                    def _():
                        out_copy(t - 1, ob).wait()
                    stage(t + 1, ob)

                pltpu.make_async_copy(
                    h_hbm.at[pl.ds(obase + t * UCH, UCH)],
                    gbuf.at[2 + tb, pl.ds(0, UCH)], hsem[tb]
                ).wait()

                @pl.loop(0, UCH, step=16)
                def _(r0):
                    dvec = dbuf[pl.ds(t * UCH + r0, 16)]
                    for i in range(16):
                        dv = jnp.full((16,), dvec[i], jnp.float32)
                        for half in (0, 16):
                            gv = gbuf[tb, r0 + i, pl.ds(half, 16)]
                            hv = gbuf[2 + tb, r0 + i, pl.ds(half, 16)]
                            gbuf[tb, r0 + i, pl.ds(half, 16)] = (
                                gv * dv + ALPHA * hv
                            )

                out_copy(t, tb).start()

        out_copy(NUC - 2, (NUC - 2) % 2).wait()
        out_copy(NUC - 1, (NUC - 1) % 2).wait()
        zero_acc().start()
        zero_acc().wait()
        plsc.subcore_barrier()

    sweep(h_hbm)
    update_sweep()

    @pl.loop(0, K - 1)
    def _(k):
        sweep(out_hbm)
        update_sweep()


_MLP_RB = 3136


def _mlp_body(x_ref, w1_ref, b1_ref, w2_ref, b2_ref, out_ref):
    h1 = lax.dot_general(
        x_ref[...], w1_ref[...], (((1,), (0,)), ((), ())),
        precision=lax.Precision.HIGHEST, preferred_element_type=jnp.float32,
    )
    h1 = jnp.maximum(h1 + b1_ref[...], 0.0)
    h2 = lax.dot_general(
        h1, w2_ref[...], (((1,), (0,)), ((), ())),
        precision=lax.Precision.HIGHEST, preferred_element_type=jnp.float32,
    )
    h2 = h2 + b2_ref[...]
    out_ref[0] = h2[:, :FH]
    out_ref[1] = h2[:, FH:]


_mlp = pl.pallas_call(
    _mlp_body,
    grid=(S // _MLP_RB,),
    in_specs=[
        pl.BlockSpec((_MLP_RB, IN_CH), lambda i: (i, 0)),
        pl.BlockSpec((IN_CH, HID_CH), lambda i: (0, 0)),
        pl.BlockSpec((1, HID_CH), lambda i: (0, 0)),
        pl.BlockSpec((HID_CH, OUT_CH), lambda i: (0, 0)),
        pl.BlockSpec((1, OUT_CH), lambda i: (0, 0)),
    ],
    out_specs=pl.BlockSpec((NC, _MLP_RB, FH), lambda i: (0, i, 0)),
    out_shape=jax.ShapeDtypeStruct((NC, S, FH), jnp.float32),
)


@jax.jit
def _appnp(x, edge_index, W1, b1, W2, b2):
    row = edge_index[0].astype(jnp.int32)
    col = edge_index[1].astype(jnp.int32)

    rowp = jnp.concatenate([row, jnp.zeros((EPAD - E,), jnp.int32)])
    colp = jnp.concatenate([col, jnp.full((EPAD - E,), N, jnp.int32)])
    r3 = rowp.reshape(NS * CPS, CH)
    c3 = colp.reshape(NS * CPS, CH)
    idx = jnp.concatenate(
        [
            jnp.stack([r3, c3], axis=1),
            jnp.stack([r3 + S, c3], axis=1),
        ],
        axis=0,
    )

    zeros = jnp.zeros((S, FH), jnp.float32)
    xpad = jnp.pad(x, ((0, S - N), (0, 0)))
    h = _mlp(xpad, W1, b1.reshape(1, HID_CH), W2, b2.reshape(1, OUT_CH))

    out = _sc_appnp(h.reshape(NC * S, FH), idx, zeros)
    return jnp.concatenate([out[:N, :], out[S:S + N, :]], axis=1)


def kernel(x, edge_index, W1, b1, W2, b2):
    return _appnp(x, edge_index, W1, b1, W2, b2)


# speedup vs baseline: 2.0526x; 2.0526x over previous
import functools

import jax
import jax.numpy as jnp
from jax import lax
from jax.experimental import pallas as pl
from jax.experimental.pallas import tpu as pltpu
from jax.experimental.pallas import tpu_sc as plsc

N = 50000
E = 800000
IN_CH = 128
HID_CH = 128
OUT_CH = 64
K = 10
ALPHA = 0.1

NC = 2
NS = 16
CH = 128
CPS = 392
EPAD = NS * CPS * CH
S = 50176
FH = 32
RPS = S // NS
UCH = 112
NUC = RPS // UCH
NB = 4
NI = 8

_MESH = plsc.VectorSubcoreMesh(
    core_axis_name="c", subcore_axis_name="s", num_cores=NC, num_subcores=NS
)


@functools.partial(
    pl.kernel,
    out_type=jax.ShapeDtypeStruct((NC * S, FH), jnp.float32),
    mesh=_MESH,
    scratch_types=[
        pltpu.VMEM_SHARED((S, FH), jnp.float32),
        pltpu.VMEM((NB, CH, FH), jnp.float32),
        pltpu.VMEM((NI, 2, CH), jnp.int32),
        pltpu.VMEM((2, UCH, FH), jnp.float32),
        pltpu.VMEM((RPS,), jnp.float32),
        [pltpu.SemaphoreType.DMA] * NB,
        [pltpu.SemaphoreType.DMA] * NB,
        [pltpu.SemaphoreType.DMA] * NI,
        pltpu.SemaphoreType.DMA,
        [pltpu.SemaphoreType.DMA] * 2,
        [pltpu.SemaphoreType.DMA] * 2,
    ],
    compiler_params=pltpu.CompilerParams(
        use_tc_tiling_on_sc=False, needs_layout_passes=False
    ),
)
def _sc_appnp(h_hbm, idx_hbm, zeros_hbm, out_hbm, acc, gbuf, ibuf, hbuf,
              dbuf, gsem, ssem, isem, zsem, osem, hsem):
    c = lax.axis_index("c")
    s = lax.axis_index("s")
    kbase = (c * NS + s) * CPS
    rbase = s * RPS
    obase = c * S + s * RPS

    def idx_copy(chunk, m):
        return pltpu.make_async_copy(idx_hbm.at[kbase + chunk], ibuf.at[m],
                                     isem[m])

    def zero_acc():
        return pltpu.make_async_copy(
            zeros_hbm.at[pl.ds(rbase, RPS)], acc.at[pl.ds(rbase, RPS)], zsem
        )

    def scat_wait(b, m):
        pltpu.make_async_copy(gbuf.at[b], acc.at[ibuf.at[m, 1]],
                              ssem[b]).wait()

    zero_acc().start()
    @pl.loop(0, CH)
    def _(i):
        gbuf[NB - 1, i, pl.ds(0, 16)] = jnp.full((16,), 1.0, jnp.float32)
        gbuf[NB - 1, i, pl.ds(16, 16)] = jnp.full((16,), 1.0, jnp.float32)

    for m in range(NI - 1):
        idx_copy(m, m).start()
    zero_acc().wait()
    plsc.subcore_barrier()

    @pl.loop(0, CPS, step=NI)
    def _(j):
        for b8 in range(NI):
            ch = j + b8
            db = b8 % NB
            pb = (b8 + NB - 1) % NB
            pi = (b8 + NI - 1) % NI
            idx_copy(ch, b8).wait()
            pltpu.async_copy(gbuf.at[NB - 1], acc.at[ibuf.at[b8, 1]],
                             ssem[db], add=True)

            @pl.when(ch > 0)
            def _():
                pltpu.make_async_copy(gbuf.at[NB - 1],
                                      acc.at[ibuf.at[pi, 1]],
                                      ssem[pb]).wait()

            @pl.when(ch + NI - 1 < CPS)
            def _():
                idx_copy(ch + NI - 1, pi).start()

    pltpu.make_async_copy(gbuf.at[NB - 1],
                          acc.at[ibuf.at[(CPS - 1) % NI, 1]],
                          ssem[(CPS - 1) % NB]).wait()
    plsc.subcore_barrier()

    @pl.loop(0, NUC)
    def _(t):
        pltpu.sync_copy(acc.at[pl.ds(rbase + t * UCH, UCH)],
                        gbuf.at[0, pl.ds(0, UCH)])

        @pl.loop(0, UCH // 16)
        def _(g):
            rows = g * 16 + lax.iota(jnp.int32, 16)
            dg = plsc.load_gather(
                gbuf, [jnp.zeros((16,), jnp.int32), rows,
                       jnp.zeros((16,), jnp.int32)]
            )
            dbuf[pl.ds(t * UCH + g * 16, 16)] = jnp.where(
                dg > 0.0, (1.0 - ALPHA) / dg, 0.0
            )

    zero_acc().start()
    zero_acc().wait()
    plsc.subcore_barrier()

    def sweep(src):
        def gath(chunk, b, m):
            return pltpu.make_async_copy(src.at[ibuf.at[m, 0]], gbuf.at[b],
                                         gsem[b])

        for m in range(NI - 1):
            idx_copy(m, m).start()
        for b in range(NB - 1):
            idx_copy(b, b).wait()
            gath(b, b, b).start()

        @pl.loop(0, CPS, step=NI)
        def _(j):
            for b8 in range(NI):
                ch = j + b8
                db = b8 % NB
                nb = (b8 + NB - 1) % NB
                ni = (b8 + NB - 1) % NI
                pi = (b8 + NI - 1) % NI

                gath(ch, db, b8).wait()
                pltpu.async_copy(gbuf.at[db], acc.at[ibuf.at[b8, 1]],
                                 ssem[db], add=True)

                nxt = ch + NB - 1
                @pl.when(nxt < CPS)
                def _():
                    @pl.when(ch > 0)
                    def _():
                        scat_wait(nb, pi)
                    idx_copy(nxt, ni).wait()
                    gath(nxt, nb, ni).start()

                @pl.when(ch + NI - 1 < CPS)
                def _():
                    idx_copy(ch + NI - 1, pi).start()

        for t in range(NB):
            chunk = CPS - NB + t
            scat_wait(chunk % NB, chunk % NI)
        plsc.subcore_barrier()

    def out_copy(t, tb):
        return pltpu.make_async_copy(
            gbuf.at[tb, pl.ds(0, UCH)],
            out_hbm.at[pl.ds(obase + t * UCH, UCH)], osem[tb]
        )

    def stage(t, tb):
        pltpu.sync_copy(acc.at[pl.ds(rbase + t * UCH, UCH)],
                        gbuf.at[tb, pl.ds(0, UCH)])
        pltpu.make_async_copy(
            h_hbm.at[pl.ds(obase + t * UCH, UCH)], hbuf.at[tb], hsem[tb]
        ).start()

    def update_sweep():
        stage(0, 0)

        @pl.loop(0, NUC, step=2)
        def _(t0):
            for b2 in range(2):
                t = t0 + b2
                tb = b2
                ob = 1 - b2

                @pl.when(t + 1 < NUC)
                def _():
                    @pl.when(t >= 1)
                    def _():
                        out_copy(t - 1, ob).wait()
                    stage(t + 1, ob)

                pltpu.make_async_copy(
                    h_hbm.at[pl.ds(obase + t * UCH, UCH)], hbuf.at[tb],
                    hsem[tb]
                ).wait()

                @pl.loop(0, UCH, step=16)
                def _(r0):
                    dvec = dbuf[pl.ds(t * UCH + r0, 16)]
                    for i in range(16):
                        dv = jnp.full((16,), dvec[i], jnp.float32)
                        for half in (0, 16):
                            gv = gbuf[tb, r0 + i, pl.ds(half, 16)]
                            hv = hbuf[tb, r0 + i, pl.ds(half, 16)]
                            gbuf[tb, r0 + i, pl.ds(half, 16)] = (
                                gv * dv + ALPHA * hv
                            )

                out_copy(t, tb).start()

        out_copy(NUC - 2, (NUC - 2) % 2).wait()
        out_copy(NUC - 1, (NUC - 1) % 2).wait()
        zero_acc().start()
        zero_acc().wait()
        plsc.subcore_barrier()

    sweep(h_hbm)
    update_sweep()

    @pl.loop(0, K - 1)
    def _(k):
        sweep(out_hbm)
        update_sweep()


_MLP_RB = 3136


def _mlp_body(x_ref, w1_ref, b1_ref, w2_ref, b2_ref, out_ref):
    h1 = lax.dot_general(
        x_ref[...], w1_ref[...], (((1,), (0,)), ((), ())),
        precision=lax.Precision.HIGHEST, preferred_element_type=jnp.float32,
    )
    h1 = jnp.maximum(h1 + b1_ref[...], 0.0)
    h2 = lax.dot_general(
        h1, w2_ref[...], (((1,), (0,)), ((), ())),
        precision=lax.Precision.HIGHEST, preferred_element_type=jnp.float32,
    )
    h2 = h2 + b2_ref[...]
    out_ref[0] = h2[:, :FH]
    out_ref[1] = h2[:, FH:]


_mlp = pl.pallas_call(
    _mlp_body,
    grid=(S // _MLP_RB,),
    in_specs=[
        pl.BlockSpec((_MLP_RB, IN_CH), lambda i: (i, 0)),
        pl.BlockSpec((IN_CH, HID_CH), lambda i: (0, 0)),
        pl.BlockSpec((1, HID_CH), lambda i: (0, 0)),
        pl.BlockSpec((HID_CH, OUT_CH), lambda i: (0, 0)),
        pl.BlockSpec((1, OUT_CH), lambda i: (0, 0)),
    ],
    out_specs=pl.BlockSpec((NC, _MLP_RB, FH), lambda i: (0, i, 0)),
    out_shape=jax.ShapeDtypeStruct((NC, S, FH), jnp.float32),
)


@jax.jit
def _appnp(x, edge_index, W1, b1, W2, b2):
    row = edge_index[0].astype(jnp.int32)
    col = edge_index[1].astype(jnp.int32)

    rowp = jnp.concatenate([row, jnp.zeros((EPAD - E,), jnp.int32)])
    colp = jnp.concatenate([col, jnp.full((EPAD - E,), N, jnp.int32)])
    r3 = rowp.reshape(NS * CPS, CH)
    c3 = colp.reshape(NS * CPS, CH)
    idx = jnp.concatenate(
        [
            jnp.stack([r3, c3], axis=1),
            jnp.stack([r3 + S, c3], axis=1),
        ],
        axis=0,
    )

    zeros = jnp.zeros((S, FH), jnp.float32)
    xpad = jnp.pad(x, ((0, S - N), (0, 0)))
    h = _mlp(xpad, W1, b1.reshape(1, HID_CH), W2, b2.reshape(1, OUT_CH))

    out = _sc_appnp(h.reshape(NC * S, FH), idx, zeros)
    return jnp.concatenate([out[:N, :], out[S:S + N, :]], axis=1)


def kernel(x, edge_index, W1, b1, W2, b2):
    return _appnp(x, edge_index, W1, b1, W2, b2)


# speedup vs baseline: 2.1180x; 1.0319x over previous
import functools

import jax
import jax.numpy as jnp
from jax import lax
from jax.experimental import pallas as pl
from jax.experimental.pallas import tpu as pltpu
from jax.experimental.pallas import tpu_sc as plsc

N = 50000
E = 800000
IN_CH = 128
HID_CH = 128
OUT_CH = 64
K = 10
ALPHA = 0.1

NC = 2
NS = 16
CH = 128
CPS = 392
EPAD = NS * CPS * CH
S = 50176
FH = 32
RPS = S // NS
UCH = 112
NUC = RPS // UCH
NB = 4
NI = 8

_MESH = plsc.VectorSubcoreMesh(
    core_axis_name="c", subcore_axis_name="s", num_cores=NC, num_subcores=NS
)


@functools.partial(
    pl.kernel,
    out_type=jax.ShapeDtypeStruct((NC, S, FH), jnp.float32),
    mesh=_MESH,
    scratch_types=[
        pltpu.VMEM_SHARED((S, FH), jnp.float32),
        pltpu.VMEM((NB, CH, FH), jnp.float32),
        pltpu.VMEM((NI, 2, CH), jnp.int32),
        pltpu.VMEM((2, UCH, FH), jnp.float32),
        pltpu.VMEM((RPS,), jnp.float32),
        [pltpu.SemaphoreType.DMA] * NB,
        [pltpu.SemaphoreType.DMA] * NB,
        [pltpu.SemaphoreType.DMA] * NI,
        pltpu.SemaphoreType.DMA,
        [pltpu.SemaphoreType.DMA] * 2,
        [pltpu.SemaphoreType.DMA] * 2,
        [pltpu.SemaphoreType.DMA] * 2,
    ],
    compiler_params=pltpu.CompilerParams(
        use_tc_tiling_on_sc=False, needs_layout_passes=False
    ),
)
def _sc_appnp(h_hbm, idx_hbm, zeros_hbm, out_hbm, acc, gbuf, ibuf, hbuf,
              dbuf, gsem, ssem, isem, zsem, osem, hsem, asem):
    c = lax.axis_index("c")
    s = lax.axis_index("s")
    kbase = s * CPS
    rbase = s * RPS
    obase = s * RPS

    def idx_copy(chunk, m):
        return pltpu.make_async_copy(idx_hbm.at[kbase + chunk], ibuf.at[m],
                                     isem[m])

    def zero_acc():
        return pltpu.make_async_copy(
            zeros_hbm.at[pl.ds(rbase, RPS)], acc.at[pl.ds(rbase, RPS)], zsem
        )

    def scat_wait(b, m):
        pltpu.make_async_copy(gbuf.at[b], acc.at[ibuf.at[m, 1]],
                              ssem[b]).wait()

    zero_acc().start()
    @pl.loop(0, CH)
    def _(i):
        gbuf[NB - 1, i, pl.ds(0, 16)] = jnp.full((16,), 1.0, jnp.float32)
        gbuf[NB - 1, i, pl.ds(16, 16)] = jnp.full((16,), 1.0, jnp.float32)

    for m in range(NI - 1):
        idx_copy(m, m).start()
    zero_acc().wait()
    plsc.subcore_barrier()

    @pl.loop(0, CPS, step=NI)
    def _(j):
        for b8 in range(NI):
            ch = j + b8
            db = b8 % NB
            pb = (b8 + NB - 1) % NB
            pi = (b8 + NI - 1) % NI
            idx_copy(ch, b8).wait()
            pltpu.async_copy(gbuf.at[NB - 1], acc.at[ibuf.at[b8, 1]],
                             ssem[db], add=True)

            @pl.when(ch > 0)
            def _():
                pltpu.make_async_copy(gbuf.at[NB - 1],
                                      acc.at[ibuf.at[pi, 1]],
                                      ssem[pb]).wait()

            @pl.when(ch + NI - 1 < CPS)
            def _():
                idx_copy(ch + NI - 1, pi).start()

    pltpu.make_async_copy(gbuf.at[NB - 1],
                          acc.at[ibuf.at[(CPS - 1) % NI, 1]],
                          ssem[(CPS - 1) % NB]).wait()
    plsc.subcore_barrier()

    @pl.loop(0, NUC)
    def _(t):
        pltpu.sync_copy(acc.at[pl.ds(rbase + t * UCH, UCH)],
                        gbuf.at[0, pl.ds(0, UCH)])

        @pl.loop(0, UCH // 16)
        def _(g):
            rows = g * 16 + lax.iota(jnp.int32, 16)
            dg = plsc.load_gather(
                gbuf, [jnp.zeros((16,), jnp.int32), rows,
                       jnp.zeros((16,), jnp.int32)]
            )
            dbuf[pl.ds(t * UCH + g * 16, 16)] = jnp.where(
                dg > 0.0, (1.0 - ALPHA) / dg, 0.0
            )

    zero_acc().start()
    zero_acc().wait()
    plsc.subcore_barrier()

    def sweep(src):
        def gath(chunk, b, m):
            return pltpu.make_async_copy(src.at[c].at[ibuf.at[m, 0]],
                                         gbuf.at[b], gsem[b])

        for m in range(NI - 1):
            idx_copy(m, m).start()
        for b in range(NB - 1):
            idx_copy(b, b).wait()
            gath(b, b, b).start()

        @pl.loop(0, CPS, step=NI)
        def _(j):
            for b8 in range(NI):
                ch = j + b8
                db = b8 % NB
                nb = (b8 + NB - 1) % NB
                ni = (b8 + NB - 1) % NI
                pi = (b8 + NI - 1) % NI

                gath(ch, db, b8).wait()
                pltpu.async_copy(gbuf.at[db], acc.at[ibuf.at[b8, 1]],
                                 ssem[db], add=True)

                nxt = ch + NB - 1
                @pl.when(nxt < CPS)
                def _():
                    @pl.when(ch > 0)
                    def _():
                        scat_wait(nb, pi)
                    idx_copy(nxt, ni).wait()
                    gath(nxt, nb, ni).start()

                @pl.when(ch + NI - 1 < CPS)
                def _():
                    idx_copy(ch + NI - 1, pi).start()

        for t in range(NB):
            chunk = CPS - NB + t
            scat_wait(chunk % NB, chunk % NI)
        plsc.subcore_barrier()

    def out_copy(t, tb):
        return pltpu.make_async_copy(
            gbuf.at[tb, pl.ds(0, UCH)],
            out_hbm.at[c].at[pl.ds(obase + t * UCH, UCH)], osem[tb]
        )

    def acc_copy(t, tb):
        return pltpu.make_async_copy(
            acc.at[pl.ds(rbase + t * UCH, UCH)],
            gbuf.at[tb, pl.ds(0, UCH)], asem[tb]
        )

    def stage(t, tb):
        acc_copy(t, tb).start()
        pltpu.make_async_copy(
            h_hbm.at[c].at[pl.ds(obase + t * UCH, UCH)], hbuf.at[tb], hsem[tb]
        ).start()

    def update_sweep():
        stage(0, 0)

        @pl.loop(0, NUC, step=2)
        def _(t0):
            for b2 in range(2):
                t = t0 + b2
                tb = b2
                ob = 1 - b2

                @pl.when(t + 1 < NUC)
                def _():
                    @pl.when(t >= 1)
                    def _():
                        out_copy(t - 1, ob).wait()
                    stage(t + 1, ob)

                pltpu.make_async_copy(
                    h_hbm.at[c].at[pl.ds(obase + t * UCH, UCH)], hbuf.at[tb],
                    hsem[tb]
                ).wait()
                acc_copy(t, tb).wait()

                @pl.loop(0, UCH, step=16)
                def _(r0):
                    dvec = dbuf[pl.ds(t * UCH + r0, 16)]
                    for i in range(16):
                        dv = jnp.full((16,), dvec[i], jnp.float32)
                        for half in (0, 16):
                            gv = gbuf[tb, r0 + i, pl.ds(half, 16)]
                            hv = hbuf[tb, r0 + i, pl.ds(half, 16)]
                            gbuf[tb, r0 + i, pl.ds(half, 16)] = (
                                gv * dv + ALPHA * hv
                            )

                out_copy(t, tb).start()

        out_copy(NUC - 2, (NUC - 2) % 2).wait()
        out_copy(NUC - 1, (NUC - 1) % 2).wait()
        zero_acc().start()
        zero_acc().wait()
        plsc.subcore_barrier()

    sweep(h_hbm)
    update_sweep()

    @pl.loop(0, K - 1)
    def _(k):
        sweep(out_hbm)
        update_sweep()


_MLP_RB = 3136


def _mlp_body(x_ref, w1_ref, b1_ref, w2_ref, b2_ref, out_ref):
    h1 = lax.dot_general(
        x_ref[...], w1_ref[...], (((1,), (0,)), ((), ())),
        precision=lax.Precision.HIGHEST, preferred_element_type=jnp.float32,
    )
    h1 = jnp.maximum(h1 + b1_ref[...], 0.0)
    h2 = lax.dot_general(
        h1, w2_ref[...], (((1,), (0,)), ((), ())),
        precision=lax.Precision.HIGHEST, preferred_element_type=jnp.float32,
    )
    h2 = h2 + b2_ref[...]
    out_ref[0] = h2[:, :FH]
    out_ref[1] = h2[:, FH:]


_mlp = pl.pallas_call(
    _mlp_body,
    grid=(S // _MLP_RB,),
    in_specs=[
        pl.BlockSpec((_MLP_RB, IN_CH), lambda i: (i, 0)),
        pl.BlockSpec((IN_CH, HID_CH), lambda i: (0, 0)),
        pl.BlockSpec((1, HID_CH), lambda i: (0, 0)),
        pl.BlockSpec((HID_CH, OUT_CH), lambda i: (0, 0)),
        pl.BlockSpec((1, OUT_CH), lambda i: (0, 0)),
    ],
    out_specs=pl.BlockSpec((NC, _MLP_RB, FH), lambda i: (0, i, 0)),
    out_shape=jax.ShapeDtypeStruct((NC, S, FH), jnp.float32),
)


@jax.jit
def _appnp(x, edge_index, W1, b1, W2, b2):
    row = edge_index[0].astype(jnp.int32)
    col = edge_index[1].astype(jnp.int32)

    rowp = jnp.concatenate([row, jnp.zeros((EPAD - E,), jnp.int32)])
    colp = jnp.concatenate([col, jnp.full((EPAD - E,), N, jnp.int32)])
    r3 = rowp.reshape(NS * CPS, CH)
    c3 = colp.reshape(NS * CPS, CH)
    idx = jnp.stack([r3, c3], axis=1)

    zeros = jnp.zeros((S, FH), jnp.float32)
    xpad = jnp.pad(x, ((0, S - N), (0, 0)))
    h = _mlp(xpad, W1, b1.reshape(1, HID_CH), W2, b2.reshape(1, OUT_CH))

    out = _sc_appnp(h, idx, zeros)
    return jnp.concatenate([out[0, :N, :], out[1, :N, :]], axis=1)


def kernel(x, edge_index, W1, b1, W2, b2):
    return _appnp(x, edge_index, W1, b1, W2, b2)


# speedup vs baseline: 2.1208x; 1.0013x over previous
import functools

import jax
import jax.numpy as jnp
from jax import lax
from jax.experimental import pallas as pl
from jax.experimental.pallas import tpu as pltpu
from jax.experimental.pallas import tpu_sc as plsc

N = 50000
E = 800000
IN_CH = 128
HID_CH = 128
OUT_CH = 64
K = 10
ALPHA = 0.1

NC = 2
NS = 16
CH = 128
CPS = 392
EPAD = NS * CPS * CH
S = 50176
FH = 32
RPS = S // NS
UCH = 112
NUC = RPS // UCH
NB = 4
NI = 8

_MESH = plsc.VectorSubcoreMesh(
    core_axis_name="c", subcore_axis_name="s", num_cores=NC, num_subcores=NS
)


@functools.partial(
    pl.kernel,
    out_type=jax.ShapeDtypeStruct((NC, S, FH), jnp.float32),
    mesh=_MESH,
    scratch_types=[
        pltpu.VMEM_SHARED((S, FH), jnp.float32),
        pltpu.VMEM((NB, CH, FH), jnp.float32),
        pltpu.VMEM((NI, 2, CH), jnp.int32),
        pltpu.VMEM((2, UCH, FH), jnp.float32),
        pltpu.VMEM((RPS,), jnp.float32),
        [pltpu.SemaphoreType.DMA] * NB,
        [pltpu.SemaphoreType.DMA] * NB,
        [pltpu.SemaphoreType.DMA] * NI,
        pltpu.SemaphoreType.DMA,
        [pltpu.SemaphoreType.DMA] * 2,
        [pltpu.SemaphoreType.DMA] * 2,
        [pltpu.SemaphoreType.DMA] * 2,
    ],
    compiler_params=pltpu.CompilerParams(
        use_tc_tiling_on_sc=False, needs_layout_passes=False
    ),
)
def _sc_appnp(h_hbm, idx_hbm, zeros_hbm, out_hbm, acc, gbuf, ibuf, hbuf,
              dbuf, gsem, ssem, isem, zsem, osem, hsem, asem):
    c = lax.axis_index("c")
    s = lax.axis_index("s")
    kbase = s * CPS
    rbase = s * RPS
    obase = s * RPS

    def idx_copy(chunk, m):
        return pltpu.make_async_copy(idx_hbm.at[kbase + chunk], ibuf.at[m],
                                     isem[m])

    def zero_acc():
        return pltpu.make_async_copy(
            zeros_hbm.at[pl.ds(rbase, RPS)], acc.at[pl.ds(rbase, RPS)], zsem
        )

    def scat_wait(b, m):
        pltpu.make_async_copy(gbuf.at[b], acc.at[ibuf.at[m, 1]],
                              ssem[b]).wait()

    zero_acc().start()
    @pl.loop(0, CH)
    def _(i):
        gbuf[NB - 1, i, pl.ds(0, 16)] = jnp.full((16,), 1.0, jnp.float32)
        gbuf[NB - 1, i, pl.ds(16, 16)] = jnp.full((16,), 1.0, jnp.float32)

    for m in range(NI - 1):
        idx_copy(m, m).start()
    zero_acc().wait()
    plsc.subcore_barrier()

    @pl.loop(0, CPS, step=NI)
    def _(j):
        for b8 in range(NI):
            ch = j + b8
            db = b8 % NB
            pb = (b8 + NB - 1) % NB
            pi = (b8 + NI - 1) % NI
            idx_copy(ch, b8).wait()
            pltpu.async_copy(gbuf.at[NB - 1], acc.at[ibuf.at[b8, 1]],
                             ssem[db], add=True)

            @pl.when(ch > 0)
            def _():
                pltpu.make_async_copy(gbuf.at[NB - 1],
                                      acc.at[ibuf.at[pi, 1]],
                                      ssem[pb]).wait()

            @pl.when(ch + NI - 1 < CPS)
            def _():
                idx_copy(ch + NI - 1, pi).start()

    pltpu.make_async_copy(gbuf.at[NB - 1],
                          acc.at[ibuf.at[(CPS - 1) % NI, 1]],
                          ssem[(CPS - 1) % NB]).wait()
    plsc.subcore_barrier()

    @pl.loop(0, NUC)
    def _(t):
        pltpu.sync_copy(acc.at[pl.ds(rbase + t * UCH, UCH)],
                        gbuf.at[0, pl.ds(0, UCH)])

        @pl.loop(0, UCH // 16)
        def _(g):
            rows = g * 16 + lax.iota(jnp.int32, 16)
            dg = plsc.load_gather(
                gbuf, [jnp.zeros((16,), jnp.int32), rows,
                       jnp.zeros((16,), jnp.int32)]
            )
            dbuf[pl.ds(t * UCH + g * 16, 16)] = jnp.where(
                dg > 0.0, (1.0 - ALPHA) / dg, 0.0
            )

    zero_acc().start()
    zero_acc().wait()
    plsc.subcore_barrier()

    def sweep(src):
        def gath(chunk, b, m):
            return pltpu.make_async_copy(src.at[c].at[ibuf.at[m, 0]],
                                         gbuf.at[b], gsem[b])

        for m in range(NI - 1):
            idx_copy(m, m).start()
        for b in range(NB - 1):
            idx_copy(b, b).wait()
            gath(b, b, b).start()

        @pl.loop(0, CPS, step=NI)
        def _(j):
            for b8 in range(NI):
                ch = j + b8
                db = b8 % NB
                nb = (b8 + NB - 1) % NB
                ni = (b8 + NB - 1) % NI
                pi = (b8 + NI - 1) % NI

                gath(ch, db, b8).wait()
                pltpu.async_copy(gbuf.at[db], acc.at[ibuf.at[b8, 1]],
                                 ssem[db], add=True)

                nxt = ch + NB - 1
                @pl.when(nxt < CPS)
                def _():
                    @pl.when(ch > 0)
                    def _():
                        scat_wait(nb, pi)
                    idx_copy(nxt, ni).wait()
                    gath(nxt, nb, ni).start()

                @pl.when(ch + NI - 1 < CPS)
                def _():
                    idx_copy(ch + NI - 1, pi).start()

        for t in range(NB):
            chunk = CPS - NB + t
            scat_wait(chunk % NB, chunk % NI)
        plsc.subcore_barrier()

    def out_copy(t, tb):
        return pltpu.make_async_copy(
            gbuf.at[tb, pl.ds(0, UCH)],
            out_hbm.at[c].at[pl.ds(obase + t * UCH, UCH)], osem[tb]
        )

    def acc_copy(t, tb):
        return pltpu.make_async_copy(
            acc.at[pl.ds(rbase + t * UCH, UCH)],
            gbuf.at[tb, pl.ds(0, UCH)], asem[tb]
        )

    def stage(t, tb):
        acc_copy(t, tb).start()
        pltpu.make_async_copy(
            h_hbm.at[c].at[pl.ds(obase + t * UCH, UCH)], hbuf.at[tb], hsem[tb]
        ).start()

    def update_sweep():
        stage(0, 0)

        @pl.loop(0, NUC, step=2)
        def _(t0):
            for b2 in range(2):
                t = t0 + b2
                tb = b2
                ob = 1 - b2

                @pl.when(t + 1 < NUC)
                def _():
                    @pl.when(t >= 1)
                    def _():
                        out_copy(t - 1, ob).wait()
                    stage(t + 1, ob)

                pltpu.make_async_copy(
                    h_hbm.at[c].at[pl.ds(obase + t * UCH, UCH)], hbuf.at[tb],
                    hsem[tb]
                ).wait()
                acc_copy(t, tb).wait()

                @pl.when(t == NUC - 1)
                def _():
                    zero_acc().start()

                @pl.loop(0, UCH, step=16)
                def _(r0):
                    dvec = dbuf[pl.ds(t * UCH + r0, 16)]
                    for i in range(16):
                        dv = jnp.full((16,), dvec[i], jnp.float32)
                        for half in (0, 16):
                            gv = gbuf[tb, r0 + i, pl.ds(half, 16)]
                            hv = hbuf[tb, r0 + i, pl.ds(half, 16)]
                            gbuf[tb, r0 + i, pl.ds(half, 16)] = (
                                gv * dv + ALPHA * hv
                            )

                out_copy(t, tb).start()

        out_copy(NUC - 2, (NUC - 2) % 2).wait()
        out_copy(NUC - 1, (NUC - 1) % 2).wait()
        zero_acc().wait()
        plsc.subcore_barrier()

    sweep(h_hbm)
    update_sweep()

    @pl.loop(0, K - 1)
    def _(k):
        sweep(out_hbm)
        update_sweep()


_MLP_RB = 3136


def _mlp_body(x_ref, w1_ref, b1_ref, w2_ref, b2_ref, out_ref):
    h1 = lax.dot_general(
        x_ref[...], w1_ref[...], (((1,), (0,)), ((), ())),
        precision=lax.Precision.HIGHEST, preferred_element_type=jnp.float32,
    )
    h1 = jnp.maximum(h1 + b1_ref[...], 0.0)
    h2 = lax.dot_general(
        h1, w2_ref[...], (((1,), (0,)), ((), ())),
        precision=lax.Precision.HIGHEST, preferred_element_type=jnp.float32,
    )
    h2 = h2 + b2_ref[...]
    out_ref[0] = h2[:, :FH]
    out_ref[1] = h2[:, FH:]


_mlp = pl.pallas_call(
    _mlp_body,
    grid=(S // _MLP_RB,),
    in_specs=[
        pl.BlockSpec((_MLP_RB, IN_CH), lambda i: (i, 0)),
        pl.BlockSpec((IN_CH, HID_CH), lambda i: (0, 0)),
        pl.BlockSpec((1, HID_CH), lambda i: (0, 0)),
        pl.BlockSpec((HID_CH, OUT_CH), lambda i: (0, 0)),
        pl.BlockSpec((1, OUT_CH), lambda i: (0, 0)),
    ],
    out_specs=pl.BlockSpec((NC, _MLP_RB, FH), lambda i: (0, i, 0)),
    out_shape=jax.ShapeDtypeStruct((NC, S, FH), jnp.float32),
)


@jax.jit
def _appnp(x, edge_index, W1, b1, W2, b2):
    row = edge_index[0].astype(jnp.int32)
    col = edge_index[1].astype(jnp.int32)

    rowp = jnp.concatenate([row, jnp.zeros((EPAD - E,), jnp.int32)])
    colp = jnp.concatenate([col, jnp.full((EPAD - E,), N, jnp.int32)])
    r3 = rowp.reshape(NS * CPS, CH)
    c3 = colp.reshape(NS * CPS, CH)
    idx = jnp.stack([r3, c3], axis=1)

    zeros = jnp.zeros((S, FH), jnp.float32)
    xpad = jnp.pad(x, ((0, S - N), (0, 0)))
    h = _mlp(xpad, W1, b1.reshape(1, HID_CH), W2, b2.reshape(1, OUT_CH))

    out = _sc_appnp(h, idx, zeros)
    return jnp.concatenate([out[0, :N, :], out[1, :N, :]], axis=1)


def kernel(x, edge_index, W1, b1, W2, b2):
    return _appnp(x, edge_index, W1, b1, W2, b2)


# speedup vs baseline: 2.1261x; 1.0025x over previous
import functools

import jax
import jax.numpy as jnp
from jax import lax
from jax.experimental import pallas as pl
from jax.experimental.pallas import tpu as pltpu
from jax.experimental.pallas import tpu_sc as plsc

N = 50000
E = 800000
IN_CH = 128
HID_CH = 128
OUT_CH = 64
K = 10
ALPHA = 0.1

NC = 2
NS = 16
CH = 128
CPS = 392
EPAD = NS * CPS * CH
S = 50176
FH = 32
RPS = S // NS
UCH = 112
NUC = RPS // UCH
NB = 4
NI = 8

_MESH = plsc.VectorSubcoreMesh(
    core_axis_name="c", subcore_axis_name="s", num_cores=NC, num_subcores=NS
)


@functools.partial(
    pl.kernel,
    out_type=jax.ShapeDtypeStruct((NC, S, FH), jnp.float32),
    mesh=_MESH,
    scratch_types=[
        pltpu.VMEM_SHARED((S, FH), jnp.float32),
        pltpu.VMEM((NB, CH, FH), jnp.float32),
        pltpu.VMEM((NI, 2, CH), jnp.int32),
        pltpu.VMEM((2, UCH, FH), jnp.float32),
        pltpu.VMEM((RPS,), jnp.float32),
        [pltpu.SemaphoreType.DMA] * NB,
        [pltpu.SemaphoreType.DMA] * NB,
        [pltpu.SemaphoreType.DMA] * NI,
        pltpu.SemaphoreType.DMA,
        [pltpu.SemaphoreType.DMA] * 2,
        [pltpu.SemaphoreType.DMA] * 2,
        [pltpu.SemaphoreType.DMA] * 2,
    ],
    compiler_params=pltpu.CompilerParams(
        use_tc_tiling_on_sc=False, needs_layout_passes=False
    ),
)
def _sc_appnp(h_hbm, idx_hbm, zeros_hbm, out_hbm, acc, gbuf, ibuf, hbuf,
              dbuf, gsem, ssem, isem, zsem, osem, hsem, asem):
    c = lax.axis_index("c")
    s = lax.axis_index("s")
    kbase = s * CPS
    rbase = s * RPS
    obase = s * RPS

    def idx_copy(chunk, m):
        return pltpu.make_async_copy(idx_hbm.at[kbase + chunk], ibuf.at[m],
                                     isem[m])

    def zero_acc():
        return pltpu.make_async_copy(
            zeros_hbm.at[pl.ds(rbase, RPS)], acc.at[pl.ds(rbase, RPS)], zsem
        )

    def scat_wait(b, m):
        pltpu.make_async_copy(gbuf.at[b], acc.at[ibuf.at[m, 1]],
                              ssem[b]).wait()

    zero_acc().start()
    @pl.loop(0, CH)
    def _(i):
        gbuf[NB - 1, i, pl.ds(0, 16)] = jnp.full((16,), 1.0, jnp.float32)
        gbuf[NB - 1, i, pl.ds(16, 16)] = jnp.full((16,), 1.0, jnp.float32)

    for m in range(4):
        idx_copy(m, m).start()
    zero_acc().wait()
    plsc.subcore_barrier()

    sem8 = list(ssem) + list(gsem)

    @pl.loop(0, CPS, step=NI)
    def _(j):
        for b8 in range(NI):
            ch = j + b8
            f4 = (b8 + 4) % NI
            idx_copy(ch, b8).wait()
            pltpu.async_copy(gbuf.at[NB - 1], acc.at[ibuf.at[b8, 1]],
                             sem8[b8], add=True)

            @pl.when(ch + 4 < CPS)
            def _():
                @pl.when(ch >= 4)
                def _():
                    pltpu.make_async_copy(gbuf.at[NB - 1],
                                          acc.at[ibuf.at[f4, 1]],
                                          sem8[f4]).wait()
                idx_copy(ch + 4, f4).start()

    for i in range(NI):
        m = (CPS - NI + i) % NI
        pltpu.make_async_copy(gbuf.at[NB - 1], acc.at[ibuf.at[m, 1]],
                              sem8[m]).wait()
    plsc.subcore_barrier()

    @pl.loop(0, NUC)
    def _(t):
        pltpu.sync_copy(acc.at[pl.ds(rbase + t * UCH, UCH)],
                        gbuf.at[0, pl.ds(0, UCH)])

        @pl.loop(0, UCH // 16)
        def _(g):
            rows = g * 16 + lax.iota(jnp.int32, 16)
            dg = plsc.load_gather(
                gbuf, [jnp.zeros((16,), jnp.int32), rows,
                       jnp.zeros((16,), jnp.int32)]
            )
            dbuf[pl.ds(t * UCH + g * 16, 16)] = jnp.where(
                dg > 0.0, (1.0 - ALPHA) / dg, 0.0
            )

    zero_acc().start()
    zero_acc().wait()
    plsc.subcore_barrier()

    def sweep(src):
        def gath(chunk, b, m):
            return pltpu.make_async_copy(src.at[c].at[ibuf.at[m, 0]],
                                         gbuf.at[b], gsem[b])

        for m in range(NI - 1):
            idx_copy(m, m).start()
        for b in range(NB - 1):
            idx_copy(b, b).wait()
            gath(b, b, b).start()

        @pl.loop(0, CPS, step=NI)
        def _(j):
            for b8 in range(NI):
                ch = j + b8
                db = b8 % NB
                nb = (b8 + NB - 1) % NB
                ni = (b8 + NB - 1) % NI
                pi = (b8 + NI - 1) % NI

                gath(ch, db, b8).wait()
                pltpu.async_copy(gbuf.at[db], acc.at[ibuf.at[b8, 1]],
                                 ssem[db], add=True)

                nxt = ch + NB - 1
                @pl.when(nxt < CPS)
                def _():
                    @pl.when(ch > 0)
                    def _():
                        scat_wait(nb, pi)
                    idx_copy(nxt, ni).wait()
                    gath(nxt, nb, ni).start()

                @pl.when(ch + NI - 1 < CPS)
                def _():
                    idx_copy(ch + NI - 1, pi).start()

        for t in range(NB):
            chunk = CPS - NB + t
            scat_wait(chunk % NB, chunk % NI)
        plsc.subcore_barrier()

    def out_copy(t, tb):
        return pltpu.make_async_copy(
            gbuf.at[tb, pl.ds(0, UCH)],
            out_hbm.at[c].at[pl.ds(obase + t * UCH, UCH)], osem[tb]
        )

    def acc_copy(t, tb):
        return pltpu.make_async_copy(
            acc.at[pl.ds(rbase + t * UCH, UCH)],
            gbuf.at[tb, pl.ds(0, UCH)], asem[tb]
        )

    def stage(t, tb):
        acc_copy(t, tb).start()
        pltpu.make_async_copy(
            h_hbm.at[c].at[pl.ds(obase + t * UCH, UCH)], hbuf.at[tb], hsem[tb]
        ).start()

    def update_sweep():
        stage(0, 0)

        @pl.loop(0, NUC, step=2)
        def _(t0):
            for b2 in range(2):
                t = t0 + b2
                tb = b2
                ob = 1 - b2

                @pl.when(t + 1 < NUC)
                def _():
                    @pl.when(t >= 1)
                    def _():
                        out_copy(t - 1, ob).wait()
                    stage(t + 1, ob)

                pltpu.make_async_copy(
                    h_hbm.at[c].at[pl.ds(obase + t * UCH, UCH)], hbuf.at[tb],
                    hsem[tb]
                ).wait()
                acc_copy(t, tb).wait()

                @pl.when(t == NUC - 1)
                def _():
                    zero_acc().start()

                @pl.loop(0, UCH, step=16)
                def _(r0):
                    dvec = dbuf[pl.ds(t * UCH + r0, 16)]
                    for i in range(16):
                        dv = jnp.full((16,), dvec[i], jnp.float32)
                        for half in (0, 16):
                            gv = gbuf[tb, r0 + i, pl.ds(half, 16)]
                            hv = hbuf[tb, r0 + i, pl.ds(half, 16)]
                            gbuf[tb, r0 + i, pl.ds(half, 16)] = (
                                gv * dv + ALPHA * hv
                            )

                out_copy(t, tb).start()

        out_copy(NUC - 2, (NUC - 2) % 2).wait()
        out_copy(NUC - 1, (NUC - 1) % 2).wait()
        zero_acc().wait()
        plsc.subcore_barrier()

    sweep(h_hbm)
    update_sweep()

    @pl.loop(0, K - 1)
    def _(k):
        sweep(out_hbm)
        update_sweep()


_MLP_RB = 3136


def _mlp_body(x_ref, w1_ref, b1_ref, w2_ref, b2_ref, out_ref):
    h1 = lax.dot_general(
        x_ref[...], w1_ref[...], (((1,), (0,)), ((), ())),
        precision=lax.Precision.HIGHEST, preferred_element_type=jnp.float32,
    )
    h1 = jnp.maximum(h1 + b1_ref[...], 0.0)
    h2 = lax.dot_general(
        h1, w2_ref[...], (((1,), (0,)), ((), ())),
        precision=lax.Precision.HIGHEST, preferred_element_type=jnp.float32,
    )
    h2 = h2 + b2_ref[...]
    out_ref[0] = h2[:, :FH]
    out_ref[1] = h2[:, FH:]


_mlp = pl.pallas_call(
    _mlp_body,
    grid=(S // _MLP_RB,),
    in_specs=[
        pl.BlockSpec((_MLP_RB, IN_CH), lambda i: (i, 0)),
        pl.BlockSpec((IN_CH, HID_CH), lambda i: (0, 0)),
        pl.BlockSpec((1, HID_CH), lambda i: (0, 0)),
        pl.BlockSpec((HID_CH, OUT_CH), lambda i: (0, 0)),
        pl.BlockSpec((1, OUT_CH), lambda i: (0, 0)),
    ],
    out_specs=pl.BlockSpec((NC, _MLP_RB, FH), lambda i: (0, i, 0)),
    out_shape=jax.ShapeDtypeStruct((NC, S, FH), jnp.float32),
)


@jax.jit
def _appnp(x, edge_index, W1, b1, W2, b2):
    row = edge_index[0].astype(jnp.int32)
    col = edge_index[1].astype(jnp.int32)

    rowp = jnp.concatenate([row, jnp.zeros((EPAD - E,), jnp.int32)])
    colp = jnp.concatenate([col, jnp.full((EPAD - E,), N, jnp.int32)])
    r3 = rowp.reshape(NS * CPS, CH)
    c3 = colp.reshape(NS * CPS, CH)
    idx = jnp.stack([r3, c3], axis=1)

    zeros = jnp.zeros((S, FH), jnp.float32)
    xpad = jnp.pad(x, ((0, S - N), (0, 0)))
    h = _mlp(xpad, W1, b1.reshape(1, HID_CH), W2, b2.reshape(1, OUT_CH))

    out = _sc_appnp(h, idx, zeros)
    return jnp.concatenate([out[0, :N, :], out[1, :N, :]], axis=1)


def kernel(x, edge_index, W1, b1, W2, b2):
    return _appnp(x, edge_index, W1, b1, W2, b2)
